# Initial kernel scaffold; baseline (speedup 1.0000x reference)
import jax
import jax.numpy as jnp
from jax.experimental import pallas as pl


def kernel(y_pred, y_true):
    raise NotImplementedError("write your pallas kernel here")


# speedup vs baseline: 1.0763x; 1.0763x over previous
import functools

import jax
import jax.numpy as jnp
from jax.experimental import pallas as pl
from jax.experimental.pallas import tpu as pltpu

EPS_ = 1e-05
LANES = 128


def _dot(x, y):
    return jax.lax.dot(x, y, precision=jax.lax.Precision.HIGHEST,
                       preferred_element_type=jnp.float32)


def _loss_body(a_ref, b_ref, out_ref):
    a = a_ref[...]
    b = b_ref[...]
    rows = a.shape[0]
    u_dim = rows // LANES
    d = a - b
    e = jnp.exp(d)

    lower_incl = (jax.lax.broadcasted_iota(jnp.int32, (LANES, LANES), 0)
                  >= jax.lax.broadcasted_iota(jnp.int32, (LANES, LANES), 1)
                  ).astype(jnp.float32)
    s_row = _dot(e, lower_incl)

    t = jnp.sum(e, axis=1, keepdims=True)
    r_i = jax.lax.broadcasted_iota(jnp.int32, (rows, u_dim), 0)
    u_i = jax.lax.broadcasted_iota(jnp.int32, (rows, u_dim), 1)
    a1 = ((r_i >> 7) == u_i).astype(jnp.float32)
    r_j = jax.lax.broadcasted_iota(jnp.int32, (rows, LANES), 0)
    v_j = jax.lax.broadcasted_iota(jnp.int32, (rows, LANES), 1)
    a2 = ((r_j & 127) == v_j).astype(jnp.float32)
    t32 = jax.lax.dot_general(a1, t * a2, (((0,), (0,)), ((), ())),
                              precision=jax.lax.Precision.HIGHEST,
                              preferred_element_type=jnp.float32)

    strict_low = (jax.lax.broadcasted_iota(jnp.int32, (LANES, LANES), 0)
                  > jax.lax.broadcasted_iota(jnp.int32, (LANES, LANES), 1)
                  ).astype(jnp.float32)
    w_t = _dot(t32, strict_low)
    g_u = jnp.sum(t32, axis=1, keepdims=True)
    strict_up_u = (jax.lax.broadcasted_iota(jnp.int32, (u_dim, u_dim), 1)
                   > jax.lax.broadcasted_iota(jnp.int32, (u_dim, u_dim), 0)
                   ).astype(jnp.float32)
    a_u = _dot(strict_up_u, g_u)
    t_excl32 = w_t + a_u
    t_excl = jnp.sum(_dot(a1, t_excl32) * a2, axis=1, keepdims=True)

    s = s_row + t_excl
    loss = jnp.log(s + EPS_) - d
    out_ref[0, 0] = jnp.sum(loss) / (rows * LANES)


def _post_sort_loss(a_half, b_half):
    rows = a_half.shape[0] // LANES
    out = pl.pallas_call(
        _loss_body,
        out_shape=jax.ShapeDtypeStruct((1, 1), jnp.float32),
        in_specs=[
            pl.BlockSpec((rows, LANES), lambda: (0, 0)),
            pl.BlockSpec((rows, LANES), lambda: (0, 0)),
        ],
        out_specs=pl.BlockSpec((1, 1), lambda: (0, 0), memory_space=pltpu.SMEM),
    )(a_half.reshape(rows, LANES), b_half.reshape(rows, LANES))
    return out.reshape(())


@jax.jit
def kernel(y_pred, y_true):
    n = y_true.shape[0]
    cn = n // 2
    idx = jnp.argsort(-y_true)
    g = jnp.take(y_pred, idx, axis=0)
    a_half = g[:cn]
    b_half = g[cn:][::-1]
    return _post_sort_loss(a_half, b_half)


# speedup vs baseline: 2.5076x; 2.3298x over previous
import functools

import jax
import jax.numpy as jnp
from jax import lax
from jax.experimental import pallas as pl
from jax.experimental.pallas import tpu as pltpu
from jax.experimental.pallas import tpu_sc as plsc

EPS_ = 1e-05
LANES = 128
N_ = 1048576
CN_ = N_ // 2
VW = 8192
NVW = N_ // VW
TRASH = 8192
PASS_SHIFTS = (0, 11, 22)
DMASK = 2047

_mesh = plsc.VectorSubcoreMesh(core_axis_name="c", subcore_axis_name="s")
_CP = pltpu.CompilerParams(needs_layout_passes=False)


def _digit_u(k, shift):
    return ((k >> jnp.uint32(shift)) & jnp.uint32(DMASK)).astype(jnp.int32)


def _digit(kf, shift):
    return _digit_u(plsc.bitcast(kf, jnp.uint32), shift)


@functools.partial(
    pl.kernel,
    out_type=[jax.ShapeDtypeStruct((N_,), jnp.float32),
              jax.ShapeDtypeStruct((NVW, 2048), jnp.int32)],
    mesh=_mesh,
    scratch_types=[
        pltpu.VMEM((VW,), jnp.float32),
        pltpu.VMEM((VW,), jnp.float32),
        pltpu.VMEM((2048,), jnp.int32),
    ],
    compiler_params=_CP,
)
def _sc_keys(y_hbm, k_hbm, h_hbm, ybuf, kbuf, tb):
    c = lax.axis_index("c")
    s = lax.axis_index("s")
    wid = s * 2 + c

    for u in range(4):
        vw = wid * 4 + u
        pltpu.sync_copy(y_hbm.at[pl.ds(vw * VW, VW)], ybuf)

        def zero_body(i, _):
            tb[pl.ds(i * 16, 16)] = jnp.zeros((16,), jnp.int32)
            return 0

        lax.fori_loop(0, 2048 // 16, zero_body, 0, unroll=4)

        def body(i, _):
            sl = pl.ds(i * 16, 16)
            y = ybuf[sl]
            bu = plsc.bitcast(y, jnp.uint32)
            neg = plsc.bitcast(y, jnp.int32) < 0
            ku = jnp.where(neg, bu, jnp.uint32(0x7FFFFFFF) - bu)
            kbuf[sl] = plsc.bitcast(ku, jnp.float32)
            d = _digit_u(ku, 0)
            cnt, lastm = plsc.scan_count(d)
            plsc.addupdate_scatter(tb, [d], cnt, mask=lastm)
            return 0

        lax.fori_loop(0, VW // 16, body, 0, unroll=2)
        pltpu.sync_copy(kbuf, k_hbm.at[pl.ds(vw * VW, VW)])
        pltpu.sync_copy(tb, h_hbm.at[vw])


def _make_hist(shift):
    @functools.partial(
        pl.kernel,
        out_type=jax.ShapeDtypeStruct((NVW, 2048), jnp.int32),
        mesh=_mesh,
        scratch_types=[
            pltpu.VMEM((VW,), jnp.float32),
            pltpu.VMEM((2048,), jnp.int32),
        ],
        compiler_params=_CP,
    )
    def _hist(k_hbm, h_hbm, kbuf, tb):
        c = lax.axis_index("c")
        s = lax.axis_index("s")

        for u in range(4):
            vw = c * 64 + s * 4 + u
            pltpu.sync_copy(k_hbm.at[pl.ds(vw * VW, VW)], kbuf)

            def zero_body(i, _):
                tb[pl.ds(i * 16, 16)] = jnp.zeros((16,), jnp.int32)
                return 0

            lax.fori_loop(0, 2048 // 16, zero_body, 0, unroll=4)

            def body(i, _):
                d = _digit(kbuf[pl.ds(i * 16, 16)], shift)
                cnt, lastm = plsc.scan_count(d)
                plsc.addupdate_scatter(tb, [d], cnt, mask=lastm)
                return 0

            lax.fori_loop(0, VW // 16, body, 0, unroll=2)
            pltpu.sync_copy(tb, h_hbm.at[vw])

    return _hist


def _make_permute(shift, last_pass, first_pass=False):
    del first_pass
    CH = VW // 2
    if last_pass:
        out_type = [jax.ShapeDtypeStruct((N_ // LANES, LANES), jnp.float32)]
    else:
        out_type = [jax.ShapeDtypeStruct((N_,), jnp.float32),
                    jax.ShapeDtypeStruct((N_,), jnp.float32)]

    scratch = (
        [pltpu.VMEM((CH,), jnp.float32) for _ in range(4)]
        + [pltpu.VMEM((CH,), jnp.int32) for _ in range(16)]
        + [pltpu.VMEM((2048,), jnp.int32) for _ in range(4)]
        + [pltpu.VMEM((2048,), jnp.int32)]
        + [pltpu.VMEM_SHARED((CN_ + TRASH,), jnp.float32),
           pltpu.SemaphoreType.DMA]
    )

    @functools.partial(
        pl.kernel,
        out_type=out_type,
        mesh=_mesh,
        scratch_types=scratch,
        compiler_params=_CP,
    )
    def _permute(k_hbm, v_hbm, pre_hbm, tot_hbm, *rest):
        if last_pass:
            vout_hbm = rest[0]
            kout_hbm = None
            rest = rest[1:]
        else:
            kout_hbm, vout_hbm = rest[:2]
            rest = rest[2:]
        kbs = rest[0:4]
        pls = rest[4:20]
        ofs = rest[20:24]
        basev = rest[24]
        ssh = rest[25]
        sem = rest[26]

        c = lax.axis_index("c")
        s = lax.axis_index("s")
        cu = c.astype(jnp.uint32)
        cn_u = jnp.uint32(CN_)
        per_tile = CN_ // 16
        lo = s * per_tile
        glo = c * CN_ + s * per_tile

        pltpu.sync_copy(tot_hbm.at[0], ofs[0])

        def base_body(i, carry):
            sl = pl.ds(i * 16, 16)
            t = ofs[0][sl]
            incl = plsc.cumsum(t)
            basev[sl] = incl - t + carry
            return carry + jnp.sum(t)

        lax.fori_loop(0, 2048 // 16, base_body, jnp.int32(0))

        for g in range(2):
            for u in range(4):
                vw = s * 8 + g * 4 + u
                pltpu.sync_copy(pre_hbm.at[vw], ofs[u])

            def init_off(i, _):
                sl = pl.ds(i * 16, 16)
                b = basev[sl]
                for u in range(4):
                    ofs[u][sl] = ofs[u][sl] + b
                return 0

            lax.fori_loop(0, 2048 // 16, init_off, 0, unroll=2)

            for h in range(2):
                for u in range(4):
                    vw = s * 8 + g * 4 + u
                    pltpu.sync_copy(
                        k_hbm.at[pl.ds(vw * VW + h * CH, CH)], kbs[u])

                def body(i, _, h=h, g=g):
                    sl = pl.ds(i * 16, 16)
                    j = (h * CH + i * 16
                         + lax.broadcasted_iota(jnp.int32, (16,), 0))
                    trash = CN_ + j
                    for u in range(4):
                        d = _digit(kbs[u][sl], shift)
                        cnt, lastm = plsc.scan_count(d)
                        bse = plsc.load_gather(ofs[u], [d])
                        pos = bse + cnt
                        plsc.store_scatter(ofs[u], [d], pos, mask=lastm)
                        if last_pass:
                            pos = jnp.where(pos < CN_, pos,
                                            (N_ - 1 + CN_) - pos)
                        my = pos.astype(jnp.uint32) - cu * cn_u
                        valid = my < cn_u
                        pls[g * 8 + u * 2 + h][sl] = jnp.where(
                            valid, my.astype(jnp.int32), trash)
                    return 0

                lax.fori_loop(0, CH // 16, body, 0, unroll=2)

                if not last_pass:
                    copies = [
                        pltpu.make_async_copy(
                            kbs[u], ssh.at[pls[g * 8 + u * 2 + h]], sem)
                        for u in range(4)]
                    for cp in copies:
                        cp.start()
                    for cp in copies:
                        cp.wait()

        if not last_pass:
            plsc.subcore_barrier()
            pltpu.sync_copy(ssh.at[pl.ds(lo, per_tile)],
                            kout_hbm.at[pl.ds(glo, per_tile)])
            plsc.subcore_barrier()

        for g in range(2):
            for h in range(2):
                for u in range(4):
                    vw = s * 8 + g * 4 + u
                    pltpu.sync_copy(
                        v_hbm.at[pl.ds(vw * VW + h * CH, CH)], kbs[u])
                copies = [
                    pltpu.make_async_copy(
                        kbs[u], ssh.at[pls[g * 8 + u * 2 + h]], sem)
                    for u in range(4)]
                for cp in copies:
                    cp.start()
                for cp in copies:
                    cp.wait()

        plsc.subcore_barrier()
        if last_pass:
            row0 = glo // LANES
            copies = [
                pltpu.make_async_copy(
                    ssh.at[pl.ds(lo + r * LANES, LANES)],
                    vout_hbm.at[row0 + r], sem)
                for r in range(per_tile // LANES)]
            for cp in copies:
                cp.start()
            for cp in copies:
                cp.wait()
        else:
            pltpu.sync_copy(ssh.at[pl.ds(lo, per_tile)],
                            vout_hbm.at[pl.ds(glo, per_tile)])

    return _permute


def _dot(x, y):
    return jax.lax.dot(x, y, precision=jax.lax.Precision.HIGHEST,
                       preferred_element_type=jnp.float32)


def _offsets_body(h_ref, pre_ref, tot_ref):
    hf = h_ref[...].astype(jnp.float32)
    nw = hf.shape[0]
    strict_low_w = (jax.lax.broadcasted_iota(jnp.int32, (nw, nw), 1)
                    < jax.lax.broadcasted_iota(jnp.int32, (nw, nw), 0)
                    ).astype(jnp.float32)
    pre = _dot(strict_low_w, hf)
    ones8 = jnp.ones((8, nw), jnp.float32)
    tot = _dot(ones8, hf)
    pre_ref[...] = (pre - 1.0).astype(jnp.int32)
    tot_ref[...] = tot.astype(jnp.int32)


def _offsets(h):
    return pl.pallas_call(
        _offsets_body,
        out_shape=[jax.ShapeDtypeStruct((NVW, 2048), jnp.int32),
                   jax.ShapeDtypeStruct((8, 2048), jnp.int32)],
        in_specs=[pl.BlockSpec((NVW, 2048), lambda: (0, 0))],
        out_specs=[pl.BlockSpec((NVW, 2048), lambda: (0, 0)),
                   pl.BlockSpec((8, 2048), lambda: (0, 0))],
    )(h)


def _loss_body(g_ref, out_ref):
    full = g_ref[...]
    half = full.shape[0] // 2
    a = full[:half, :]
    b = full[half:, :]
    rows = a.shape[0]
    u_dim = rows // LANES
    d = a - b
    e = jnp.exp(d)

    lower_incl = (jax.lax.broadcasted_iota(jnp.int32, (LANES, LANES), 0)
                  >= jax.lax.broadcasted_iota(jnp.int32, (LANES, LANES), 1)
                  ).astype(jnp.float32)
    s_row = _dot(e, lower_incl)

    t = jnp.sum(e, axis=1, keepdims=True)
    r_i = jax.lax.broadcasted_iota(jnp.int32, (rows, u_dim), 0)
    u_i = jax.lax.broadcasted_iota(jnp.int32, (rows, u_dim), 1)
    a1 = ((r_i >> 7) == u_i).astype(jnp.float32)
    r_j = jax.lax.broadcasted_iota(jnp.int32, (rows, LANES), 0)
    v_j = jax.lax.broadcasted_iota(jnp.int32, (rows, LANES), 1)
    a2 = ((r_j & 127) == v_j).astype(jnp.float32)
    t32 = jax.lax.dot_general(a1, t * a2, (((0,), (0,)), ((), ())),
                              precision=jax.lax.Precision.HIGHEST,
                              preferred_element_type=jnp.float32)

    strict_low = (jax.lax.broadcasted_iota(jnp.int32, (LANES, LANES), 0)
                  > jax.lax.broadcasted_iota(jnp.int32, (LANES, LANES), 1)
                  ).astype(jnp.float32)
    w_t = _dot(t32, strict_low)
    g_u = jnp.sum(t32, axis=1, keepdims=True)
    strict_up_u = (jax.lax.broadcasted_iota(jnp.int32, (u_dim, u_dim), 1)
                   > jax.lax.broadcasted_iota(jnp.int32, (u_dim, u_dim), 0)
                   ).astype(jnp.float32)
    a_u = _dot(strict_up_u, g_u)
    t_excl32 = w_t + a_u
    t_excl = jnp.sum(_dot(a1, t_excl32) * a2, axis=1, keepdims=True)

    s = s_row + t_excl
    loss = jnp.log(s + EPS_) - d
    out_ref[0, 0] = jnp.sum(loss) / (rows * LANES)


def _post_sort_loss(g2d):
    rows = g2d.shape[0]
    out = pl.pallas_call(
        _loss_body,
        out_shape=jax.ShapeDtypeStruct((1, 1), jnp.float32),
        in_specs=[pl.BlockSpec((rows, LANES), lambda: (0, 0))],
        out_specs=pl.BlockSpec((1, 1), lambda: (0, 0), memory_space=pltpu.SMEM),
    )(g2d)
    return out.reshape(())


_perm1 = _make_permute(PASS_SHIFTS[0], False)
_perm2 = _make_permute(PASS_SHIFTS[1], False)
_perm3 = _make_permute(PASS_SHIFTS[2], True)
_hist2 = _make_hist(PASS_SHIFTS[1])
_hist3 = _make_hist(PASS_SHIFTS[2])


@jax.jit
def kernel(y_pred, y_true):
    k0, h1 = _sc_keys(y_true)
    pre1, tot1 = _offsets(h1)
    k1, v1 = _perm1(k0, y_pred, pre1, tot1)
    h2 = _hist2(k1)
    pre2, tot2 = _offsets(h2)
    k2, v2 = _perm2(k1, v1, pre2, tot2)
    h3 = _hist3(k2)
    pre3, tot3 = _offsets(h3)
    (g_u,) = _perm3(k2, v2, pre3, tot3)
    return _post_sort_loss(g_u)


# speedup vs baseline: 2.5592x; 1.0206x over previous
import functools

import jax
import jax.numpy as jnp
from jax import lax
from jax.experimental import pallas as pl
from jax.experimental.pallas import tpu as pltpu
from jax.experimental.pallas import tpu_sc as plsc

EPS_ = 1e-05
LANES = 128
N_ = 1048576
CN_ = N_ // 2
VW = 8192
NVW = N_ // VW
TRASH = 8192
PASS_SHIFTS = (0, 11, 22)
DMASK = 2047

_mesh = plsc.VectorSubcoreMesh(core_axis_name="c", subcore_axis_name="s")
_CP = pltpu.CompilerParams(needs_layout_passes=False)


def _digit_u(k, shift):
    return ((k >> jnp.uint32(shift)) & jnp.uint32(DMASK)).astype(jnp.int32)


def _digit(kf, shift):
    return _digit_u(plsc.bitcast(kf, jnp.uint32), shift)


@functools.partial(
    pl.kernel,
    out_type=[jax.ShapeDtypeStruct((N_,), jnp.float32),
              jax.ShapeDtypeStruct((NVW, 2048), jnp.int32)],
    mesh=_mesh,
    scratch_types=[
        pltpu.VMEM((VW,), jnp.float32),
        pltpu.VMEM((VW,), jnp.float32),
        pltpu.VMEM((2048,), jnp.int32),
    ],
    compiler_params=_CP,
)
def _sc_keys(y_hbm, k_hbm, h_hbm, ybuf, kbuf, tb):
    c = lax.axis_index("c")
    s = lax.axis_index("s")
    wid = s * 2 + c

    for u in range(4):
        vw = wid * 4 + u
        pltpu.sync_copy(y_hbm.at[pl.ds(vw * VW, VW)], ybuf)

        def zero_body(i, _):
            tb[pl.ds(i * 16, 16)] = jnp.zeros((16,), jnp.int32)
            return 0

        lax.fori_loop(0, 2048 // 16, zero_body, 0, unroll=4)

        def body(i, _):
            sl = pl.ds(i * 16, 16)
            y = ybuf[sl]
            bu = plsc.bitcast(y, jnp.uint32)
            neg = plsc.bitcast(y, jnp.int32) < 0
            ku = jnp.where(neg, bu, jnp.uint32(0x7FFFFFFF) - bu)
            kbuf[sl] = plsc.bitcast(ku, jnp.float32)
            d = _digit_u(ku, 0)
            cnt, lastm = plsc.scan_count(d)
            plsc.addupdate_scatter(tb, [d], cnt, mask=lastm)
            return 0

        lax.fori_loop(0, VW // 16, body, 0, unroll=2)
        pltpu.sync_copy(kbuf, k_hbm.at[pl.ds(vw * VW, VW)])
        pltpu.sync_copy(tb, h_hbm.at[vw])


def _make_hist(shift):
    @functools.partial(
        pl.kernel,
        out_type=jax.ShapeDtypeStruct((NVW, 2048), jnp.int32),
        mesh=_mesh,
        scratch_types=[
            pltpu.VMEM((VW,), jnp.float32),
            pltpu.VMEM((2048,), jnp.int32),
        ],
        compiler_params=_CP,
    )
    def _hist(k_hbm, h_hbm, kbuf, tb):
        c = lax.axis_index("c")
        s = lax.axis_index("s")

        for u in range(4):
            vw = c * 64 + s * 4 + u
            pltpu.sync_copy(k_hbm.at[pl.ds(vw * VW, VW)], kbuf)

            def zero_body(i, _):
                tb[pl.ds(i * 16, 16)] = jnp.zeros((16,), jnp.int32)
                return 0

            lax.fori_loop(0, 2048 // 16, zero_body, 0, unroll=4)

            def body(i, _):
                d = _digit(kbuf[pl.ds(i * 16, 16)], shift)
                cnt, lastm = plsc.scan_count(d)
                plsc.addupdate_scatter(tb, [d], cnt, mask=lastm)
                return 0

            lax.fori_loop(0, VW // 16, body, 0, unroll=2)
            pltpu.sync_copy(tb, h_hbm.at[vw])

    return _hist


def _make_permute(shift, last_pass, next_shift=None):
    CH = VW // 2
    if last_pass:
        out_type = [jax.ShapeDtypeStruct((N_ // LANES, LANES), jnp.float32)]
    else:
        out_type = [jax.ShapeDtypeStruct((N_,), jnp.float32),
                    jax.ShapeDtypeStruct((N_,), jnp.float32),
                    jax.ShapeDtypeStruct((NVW, 2048), jnp.int32)]

    scratch = (
        [pltpu.VMEM((CH,), jnp.float32) for _ in range(4)]
        + [pltpu.VMEM((CH,), jnp.int32) for _ in range(16)]
        + [pltpu.VMEM((2048,), jnp.int32) for _ in range(4)]
        + [pltpu.VMEM((2048,), jnp.int32)]
        + [pltpu.VMEM_SHARED((CN_ + TRASH,), jnp.float32),
           pltpu.SemaphoreType.DMA]
    )

    @functools.partial(
        pl.kernel,
        out_type=out_type,
        mesh=_mesh,
        scratch_types=scratch,
        compiler_params=_CP,
    )
    def _permute(k_hbm, v_hbm, pre_hbm, tot_hbm, *rest):
        if last_pass:
            vout_hbm = rest[0]
            kout_hbm = hnext_hbm = None
            rest = rest[1:]
        else:
            kout_hbm, vout_hbm, hnext_hbm = rest[:3]
            rest = rest[3:]
        kbs = rest[0:4]
        pls = rest[4:20]
        ofs = rest[20:24]
        basev = rest[24]
        ssh = rest[25]
        sem = rest[26]

        c = lax.axis_index("c")
        s = lax.axis_index("s")
        cu = c.astype(jnp.uint32)
        cn_u = jnp.uint32(CN_)
        per_tile = CN_ // 16
        lo = s * per_tile
        glo = c * CN_ + s * per_tile

        pltpu.sync_copy(tot_hbm.at[0], ofs[0])

        def base_body(i, carry):
            sl = pl.ds(i * 16, 16)
            t = ofs[0][sl]
            incl = plsc.cumsum(t)
            basev[sl] = incl - t + carry
            return carry + jnp.sum(t)

        lax.fori_loop(0, 2048 // 16, base_body, jnp.int32(0))

        for g in range(2):
            for u in range(4):
                vw = s * 8 + g * 4 + u
                pltpu.sync_copy(pre_hbm.at[vw], ofs[u])

            def init_off(i, _):
                sl = pl.ds(i * 16, 16)
                b = basev[sl]
                for u in range(4):
                    ofs[u][sl] = ofs[u][sl] + b
                return 0

            lax.fori_loop(0, 2048 // 16, init_off, 0, unroll=2)

            for h in range(2):
                for u in range(4):
                    vw = s * 8 + g * 4 + u
                    pltpu.sync_copy(
                        k_hbm.at[pl.ds(vw * VW + h * CH, CH)], kbs[u])

                def body(i, _, h=h, g=g):
                    sl = pl.ds(i * 16, 16)
                    j = (h * CH + i * 16
                         + lax.broadcasted_iota(jnp.int32, (16,), 0))
                    trash = CN_ + j
                    for u in range(4):
                        d = _digit(kbs[u][sl], shift)
                        cnt, lastm = plsc.scan_count(d)
                        bse = plsc.load_gather(ofs[u], [d])
                        pos = bse + cnt
                        plsc.store_scatter(ofs[u], [d], pos, mask=lastm)
                        if last_pass:
                            pos = jnp.where(pos < CN_, pos,
                                            (N_ - 1 + CN_) - pos)
                        my = pos.astype(jnp.uint32) - cu * cn_u
                        valid = my < cn_u
                        pls[g * 8 + u * 2 + h][sl] = jnp.where(
                            valid, my.astype(jnp.int32), trash)
                    return 0

                lax.fori_loop(0, CH // 16, body, 0, unroll=2)

                if not last_pass:
                    copies = [
                        pltpu.make_async_copy(
                            kbs[u], ssh.at[pls[g * 8 + u * 2 + h]], sem)
                        for u in range(4)]
                    for cp in copies:
                        cp.start()
                    for cp in copies:
                        cp.wait()

        if not last_pass:
            plsc.subcore_barrier()
            pltpu.sync_copy(ssh.at[pl.ds(lo, per_tile)],
                            kout_hbm.at[pl.ds(glo, per_tile)])

            def zero_body(i, _):
                sl = pl.ds(i * 16, 16)
                z = jnp.zeros((16,), jnp.int32)
                for u in range(4):
                    ofs[u][sl] = z
                return 0

            lax.fori_loop(0, 2048 // 16, zero_body, 0, unroll=2)
            for u in range(4):
                for h in range(2):
                    pltpu.sync_copy(
                        ssh.at[pl.ds(lo + u * VW + h * CH, CH)], kbs[u])

                    def hist_body(i, _, u=u):
                        d = _digit(kbs[u][pl.ds(i * 16, 16)], next_shift)
                        cnt, lastm = plsc.scan_count(d)
                        plsc.addupdate_scatter(ofs[u], [d], cnt, mask=lastm)
                        return 0

                    lax.fori_loop(0, CH // 16, hist_body, 0, unroll=2)
            vw0 = glo // VW
            for u in range(4):
                pltpu.sync_copy(ofs[u], hnext_hbm.at[vw0 + u])

            plsc.subcore_barrier()

        for g in range(2):
            for h in range(2):
                for u in range(4):
                    vw = s * 8 + g * 4 + u
                    pltpu.sync_copy(
                        v_hbm.at[pl.ds(vw * VW + h * CH, CH)], kbs[u])
                copies = [
                    pltpu.make_async_copy(
                        kbs[u], ssh.at[pls[g * 8 + u * 2 + h]], sem)
                    for u in range(4)]
                for cp in copies:
                    cp.start()
                for cp in copies:
                    cp.wait()

        plsc.subcore_barrier()
        if last_pass:
            row0 = glo // LANES
            copies = [
                pltpu.make_async_copy(
                    ssh.at[pl.ds(lo + r * LANES, LANES)],
                    vout_hbm.at[row0 + r], sem)
                for r in range(per_tile // LANES)]
            for cp in copies:
                cp.start()
            for cp in copies:
                cp.wait()
        else:
            pltpu.sync_copy(ssh.at[pl.ds(lo, per_tile)],
                            vout_hbm.at[pl.ds(glo, per_tile)])

    return _permute


def _dot(x, y):
    return jax.lax.dot(x, y, precision=jax.lax.Precision.HIGHEST,
                       preferred_element_type=jnp.float32)


def _offsets_body(h_ref, pre_ref, tot_ref):
    hf = h_ref[...].astype(jnp.float32)
    nw = hf.shape[0]
    strict_low_w = (jax.lax.broadcasted_iota(jnp.int32, (nw, nw), 1)
                    < jax.lax.broadcasted_iota(jnp.int32, (nw, nw), 0)
                    ).astype(jnp.float32)
    pre = _dot(strict_low_w, hf)
    ones8 = jnp.ones((8, nw), jnp.float32)
    tot = _dot(ones8, hf)
    pre_ref[...] = (pre - 1.0).astype(jnp.int32)
    tot_ref[...] = tot.astype(jnp.int32)


def _offsets(h):
    return pl.pallas_call(
        _offsets_body,
        out_shape=[jax.ShapeDtypeStruct((NVW, 2048), jnp.int32),
                   jax.ShapeDtypeStruct((8, 2048), jnp.int32)],
        in_specs=[pl.BlockSpec((NVW, 2048), lambda: (0, 0))],
        out_specs=[pl.BlockSpec((NVW, 2048), lambda: (0, 0)),
                   pl.BlockSpec((8, 2048), lambda: (0, 0))],
    )(h)


def _loss_body(g_ref, out_ref):
    full = g_ref[...]
    half = full.shape[0] // 2
    a = full[:half, :]
    b = full[half:, :]
    rows = a.shape[0]
    u_dim = rows // LANES
    d = a - b
    e = jnp.exp(d)

    lower_incl = (jax.lax.broadcasted_iota(jnp.int32, (LANES, LANES), 0)
                  >= jax.lax.broadcasted_iota(jnp.int32, (LANES, LANES), 1)
                  ).astype(jnp.float32)
    s_row = _dot(e, lower_incl)

    t = jnp.sum(e, axis=1, keepdims=True)
    r_i = jax.lax.broadcasted_iota(jnp.int32, (rows, u_dim), 0)
    u_i = jax.lax.broadcasted_iota(jnp.int32, (rows, u_dim), 1)
    a1 = ((r_i >> 7) == u_i).astype(jnp.float32)
    r_j = jax.lax.broadcasted_iota(jnp.int32, (rows, LANES), 0)
    v_j = jax.lax.broadcasted_iota(jnp.int32, (rows, LANES), 1)
    a2 = ((r_j & 127) == v_j).astype(jnp.float32)
    t32 = jax.lax.dot_general(a1, t * a2, (((0,), (0,)), ((), ())),
                              precision=jax.lax.Precision.HIGHEST,
                              preferred_element_type=jnp.float32)

    strict_low = (jax.lax.broadcasted_iota(jnp.int32, (LANES, LANES), 0)
                  > jax.lax.broadcasted_iota(jnp.int32, (LANES, LANES), 1)
                  ).astype(jnp.float32)
    w_t = _dot(t32, strict_low)
    g_u = jnp.sum(t32, axis=1, keepdims=True)
    strict_up_u = (jax.lax.broadcasted_iota(jnp.int32, (u_dim, u_dim), 1)
                   > jax.lax.broadcasted_iota(jnp.int32, (u_dim, u_dim), 0)
                   ).astype(jnp.float32)
    a_u = _dot(strict_up_u, g_u)
    t_excl32 = w_t + a_u
    t_excl = jnp.sum(_dot(a1, t_excl32) * a2, axis=1, keepdims=True)

    s = s_row + t_excl
    loss = jnp.log(s + EPS_) - d
    out_ref[0, 0] = jnp.sum(loss) / (rows * LANES)


def _post_sort_loss(g2d):
    rows = g2d.shape[0]
    out = pl.pallas_call(
        _loss_body,
        out_shape=jax.ShapeDtypeStruct((1, 1), jnp.float32),
        in_specs=[pl.BlockSpec((rows, LANES), lambda: (0, 0))],
        out_specs=pl.BlockSpec((1, 1), lambda: (0, 0), memory_space=pltpu.SMEM),
    )(g2d)
    return out.reshape(())


_perm1 = _make_permute(PASS_SHIFTS[0], False, next_shift=PASS_SHIFTS[1])
_perm2 = _make_permute(PASS_SHIFTS[1], False, next_shift=PASS_SHIFTS[2])
_perm3 = _make_permute(PASS_SHIFTS[2], True)


@jax.jit
def kernel(y_pred, y_true):
    k0, h1 = _sc_keys(y_true)
    pre1, tot1 = _offsets(h1)
    k1, v1, h2 = _perm1(k0, y_pred, pre1, tot1)
    pre2, tot2 = _offsets(h2)
    k2, v2, h3 = _perm2(k1, v1, pre2, tot2)
    pre3, tot3 = _offsets(h3)
    (g_u,) = _perm3(k2, v2, pre3, tot3)
    return _post_sort_loss(g_u)


# speedup vs baseline: 2.9612x; 1.1571x over previous
import functools

import jax
import jax.numpy as jnp
from jax import lax
from jax.experimental import pallas as pl
from jax.experimental.pallas import tpu as pltpu
from jax.experimental.pallas import tpu_sc as plsc

EPS_ = 1e-05
LANES = 128
N_ = 1048576
CN_ = N_ // 2
VW = 8192
NVW = N_ // VW
TRASH = 8192
PASS_SHIFTS = (0, 11, 22)
DMASK = 2047

_mesh = plsc.VectorSubcoreMesh(core_axis_name="c", subcore_axis_name="s")
_CP = pltpu.CompilerParams(needs_layout_passes=False)


def _digit_u(k, shift):
    return ((k >> jnp.uint32(shift)) & jnp.uint32(DMASK)).astype(jnp.int32)


def _digit(kf, shift):
    return _digit_u(plsc.bitcast(kf, jnp.uint32), shift)


@functools.partial(
    pl.kernel,
    out_type=[jax.ShapeDtypeStruct((N_,), jnp.float32),
              jax.ShapeDtypeStruct((NVW, 2048), jnp.int32)],
    mesh=_mesh,
    scratch_types=[
        pltpu.VMEM((VW,), jnp.float32),
        pltpu.VMEM((VW,), jnp.float32),
        pltpu.VMEM((2048,), jnp.int32),
    ],
    compiler_params=_CP,
)
def _sc_keys(y_hbm, k_hbm, h_hbm, ybuf, kbuf, tb):
    c = lax.axis_index("c")
    s = lax.axis_index("s")
    wid = s * 2 + c

    for u in range(4):
        vw = wid * 4 + u
        pltpu.sync_copy(y_hbm.at[pl.ds(vw * VW, VW)], ybuf)

        def zero_body(i, _):
            tb[pl.ds(i * 16, 16)] = jnp.zeros((16,), jnp.int32)
            return 0

        lax.fori_loop(0, 2048 // 16, zero_body, 0, unroll=4)

        def body(i, _):
            sl = pl.ds(i * 16, 16)
            y = ybuf[sl]
            bu = plsc.bitcast(y, jnp.uint32)
            neg = plsc.bitcast(y, jnp.int32) < 0
            ku = jnp.where(neg, bu, jnp.uint32(0x7FFFFFFF) - bu)
            kbuf[sl] = plsc.bitcast(ku, jnp.float32)
            d = _digit_u(ku, 0)
            cnt, lastm = plsc.scan_count(d)
            plsc.addupdate_scatter(tb, [d], cnt, mask=lastm)
            return 0

        lax.fori_loop(0, VW // 16, body, 0, unroll=2)
        pltpu.sync_copy(kbuf, k_hbm.at[pl.ds(vw * VW, VW)])
        pltpu.sync_copy(tb, h_hbm.at[vw])


def _make_hist(shift):
    @functools.partial(
        pl.kernel,
        out_type=jax.ShapeDtypeStruct((NVW, 2048), jnp.int32),
        mesh=_mesh,
        scratch_types=[
            pltpu.VMEM((VW,), jnp.float32),
            pltpu.VMEM((2048,), jnp.int32),
        ],
        compiler_params=_CP,
    )
    def _hist(k_hbm, h_hbm, kbuf, tb):
        c = lax.axis_index("c")
        s = lax.axis_index("s")

        for u in range(4):
            vw = c * 64 + s * 4 + u
            pltpu.sync_copy(k_hbm.at[pl.ds(vw * VW, VW)], kbuf)

            def zero_body(i, _):
                tb[pl.ds(i * 16, 16)] = jnp.zeros((16,), jnp.int32)
                return 0

            lax.fori_loop(0, 2048 // 16, zero_body, 0, unroll=4)

            def body(i, _):
                d = _digit(kbuf[pl.ds(i * 16, 16)], shift)
                cnt, lastm = plsc.scan_count(d)
                plsc.addupdate_scatter(tb, [d], cnt, mask=lastm)
                return 0

            lax.fori_loop(0, VW // 16, body, 0, unroll=2)
            pltpu.sync_copy(tb, h_hbm.at[vw])

    return _hist


def _make_permute(shift, last_pass, next_shift=None):
    CH = VW // 4
    if last_pass:
        out_type = [jax.ShapeDtypeStruct((N_ // LANES, LANES), jnp.float32)]
    else:
        out_type = [jax.ShapeDtypeStruct((N_,), jnp.float32),
                    jax.ShapeDtypeStruct((N_,), jnp.float32),
                    jax.ShapeDtypeStruct((NVW, 2048), jnp.int32)]

    scratch = (
        [pltpu.VMEM((CH,), jnp.float32) for _ in range(8)]
        + [pltpu.VMEM((CH,), jnp.int32) for _ in range(32)]
        + [pltpu.VMEM((2048,), jnp.int32) for _ in range(4)]
        + [pltpu.VMEM((2048,), jnp.int32)]
        + [pltpu.VMEM_SHARED((CN_ + TRASH,), jnp.float32),
           pltpu.SemaphoreType.DMA, pltpu.SemaphoreType.DMA]
    )

    @functools.partial(
        pl.kernel,
        out_type=out_type,
        mesh=_mesh,
        scratch_types=scratch,
        compiler_params=_CP,
    )
    def _permute(k_hbm, v_hbm, pre_hbm, tot_hbm, *rest):
        if last_pass:
            vout_hbm = rest[0]
            kout_hbm = hnext_hbm = None
            rest = rest[1:]
        else:
            kout_hbm, vout_hbm, hnext_hbm = rest[:3]
            rest = rest[3:]
        kbs = rest[0:8]
        pls = rest[8:40]
        ofs = rest[40:44]
        basev = rest[44]
        ssh = rest[45]
        sem = rest[46]
        sem2 = rest[47]

        c = lax.axis_index("c")
        s = lax.axis_index("s")
        cu = c.astype(jnp.uint32)
        cn_u = jnp.uint32(CN_)
        per_tile = CN_ // 16
        lo = s * per_tile
        glo = c * CN_ + s * per_tile

        pltpu.sync_copy(tot_hbm.at[0], ofs[0])

        def base_body(i, carry):
            sl = pl.ds(i * 16, 16)
            t = ofs[0][sl]
            incl = plsc.cumsum(t)
            basev[sl] = incl - t + carry
            return carry + jnp.sum(t)

        lax.fori_loop(0, 2048 // 16, base_body, jnp.int32(0))

        def src_off(p, u):
            vw = s * 8 + (p // 4) * 4 + u
            return vw * VW + (p % 4) * CH

        def start_loads(src_hbm, p):
            objs = []
            for u in range(4):
                cpo = pltpu.make_async_copy(
                    src_hbm.at[pl.ds(src_off(p, u), CH)],
                    kbs[(p % 2) * 4 + u], sem2)
                cpo.start()
                objs.append(cpo)
            return objs

        def start_scats(p):
            objs = []
            for u in range(4):
                cpo = pltpu.make_async_copy(
                    kbs[(p % 2) * 4 + u], ssh.at[pls[p * 4 + u]], sem)
                cpo.start()
                objs.append(cpo)
            return objs

        loads = {0: start_loads(k_hbm, 0)}
        scats = {}
        for p in range(8):
            if p % 4 == 0:
                for u in range(4):
                    vw = s * 8 + (p // 4) * 4 + u
                    pltpu.sync_copy(pre_hbm.at[vw], ofs[u])

                def init_off(i, _):
                    sl = pl.ds(i * 16, 16)
                    b = basev[sl]
                    for u in range(4):
                        ofs[u][sl] = ofs[u][sl] + b
                    return 0

                lax.fori_loop(0, 2048 // 16, init_off, 0, unroll=2)

            for cpo in loads[p]:
                cpo.wait()

            def body(i, _, p=p):
                sl = pl.ds(i * 16, 16)
                j = ((p % 4) * CH + i * 16
                     + lax.broadcasted_iota(jnp.int32, (16,), 0))
                trash = CN_ + j
                for u in range(4):
                    d = _digit(kbs[(p % 2) * 4 + u][sl], shift)
                    cnt, lastm = plsc.scan_count(d)
                    bse = plsc.load_gather(ofs[u], [d])
                    pos = bse + cnt
                    plsc.store_scatter(ofs[u], [d], pos, mask=lastm)
                    if last_pass:
                        pos = jnp.where(pos < CN_, pos,
                                        (N_ - 1 + CN_) - pos)
                    my = pos.astype(jnp.uint32) - cu * cn_u
                    valid = my < cn_u
                    pls[p * 4 + u][sl] = jnp.where(
                        valid, my.astype(jnp.int32), trash)
                return 0

            lax.fori_loop(0, CH // 16, body, 0, unroll=2)

            if not last_pass:
                scats[p] = start_scats(p)
                if p - 1 in scats:
                    for cpo in scats[p - 1]:
                        cpo.wait()
            if p + 1 < 8:
                loads[p + 1] = start_loads(k_hbm, p + 1)
        if not last_pass:
            for cpo in scats[7]:
                cpo.wait()

        if not last_pass:
            plsc.subcore_barrier()
            pltpu.sync_copy(ssh.at[pl.ds(lo, per_tile)],
                            kout_hbm.at[pl.ds(glo, per_tile)])

            def zero_body(i, _):
                sl = pl.ds(i * 16, 16)
                z = jnp.zeros((16,), jnp.int32)
                for u in range(4):
                    ofs[u][sl] = z
                return 0

            lax.fori_loop(0, 2048 // 16, zero_body, 0, unroll=2)
            for u in range(4):
                for h in range(4):
                    pltpu.sync_copy(
                        ssh.at[pl.ds(lo + u * VW + h * CH, CH)], kbs[u])

                    def hist_body(i, _, u=u):
                        d = _digit(kbs[u][pl.ds(i * 16, 16)], next_shift)
                        cnt, lastm = plsc.scan_count(d)
                        plsc.addupdate_scatter(ofs[u], [d], cnt, mask=lastm)
                        return 0

                    lax.fori_loop(0, CH // 16, hist_body, 0, unroll=2)
            vw0 = glo // VW
            for u in range(4):
                pltpu.sync_copy(ofs[u], hnext_hbm.at[vw0 + u])

            plsc.subcore_barrier()

        loads = {0: start_loads(v_hbm, 0)}
        scats = {}
        for p in range(8):
            for cpo in loads[p]:
                cpo.wait()
            scats[p] = start_scats(p)
            if p - 1 in scats:
                for cpo in scats[p - 1]:
                    cpo.wait()
            if p + 1 < 8:
                loads[p + 1] = start_loads(v_hbm, p + 1)
        for cpo in scats[7]:
            cpo.wait()

        plsc.subcore_barrier()
        if last_pass:
            row0 = glo // LANES
            copies = [
                pltpu.make_async_copy(
                    ssh.at[pl.ds(lo + r * LANES, LANES)],
                    vout_hbm.at[row0 + r], sem)
                for r in range(per_tile // LANES)]
            for cp in copies:
                cp.start()
            for cp in copies:
                cp.wait()
        else:
            pltpu.sync_copy(ssh.at[pl.ds(lo, per_tile)],
                            vout_hbm.at[pl.ds(glo, per_tile)])

    return _permute


def _dot(x, y):
    return jax.lax.dot(x, y, precision=jax.lax.Precision.HIGHEST,
                       preferred_element_type=jnp.float32)


def _offsets_body(h_ref, pre_ref, tot_ref):
    hf = h_ref[...].astype(jnp.float32)
    nw = hf.shape[0]
    strict_low_w = (jax.lax.broadcasted_iota(jnp.int32, (nw, nw), 1)
                    < jax.lax.broadcasted_iota(jnp.int32, (nw, nw), 0)
                    ).astype(jnp.float32)
    pre = _dot(strict_low_w, hf)
    ones8 = jnp.ones((8, nw), jnp.float32)
    tot = _dot(ones8, hf)
    pre_ref[...] = (pre - 1.0).astype(jnp.int32)
    tot_ref[...] = tot.astype(jnp.int32)


def _offsets(h):
    return pl.pallas_call(
        _offsets_body,
        out_shape=[jax.ShapeDtypeStruct((NVW, 2048), jnp.int32),
                   jax.ShapeDtypeStruct((8, 2048), jnp.int32)],
        in_specs=[pl.BlockSpec((NVW, 2048), lambda: (0, 0))],
        out_specs=[pl.BlockSpec((NVW, 2048), lambda: (0, 0)),
                   pl.BlockSpec((8, 2048), lambda: (0, 0))],
    )(h)


def _loss_body(g_ref, out_ref):
    full = g_ref[...]
    half = full.shape[0] // 2
    a = full[:half, :]
    b = full[half:, :]
    rows = a.shape[0]
    u_dim = rows // LANES
    d = a - b
    e = jnp.exp(d)

    lower_incl = (jax.lax.broadcasted_iota(jnp.int32, (LANES, LANES), 0)
                  >= jax.lax.broadcasted_iota(jnp.int32, (LANES, LANES), 1)
                  ).astype(jnp.float32)
    s_row = _dot(e, lower_incl)

    t = jnp.sum(e, axis=1, keepdims=True)
    r_i = jax.lax.broadcasted_iota(jnp.int32, (rows, u_dim), 0)
    u_i = jax.lax.broadcasted_iota(jnp.int32, (rows, u_dim), 1)
    a1 = ((r_i >> 7) == u_i).astype(jnp.float32)
    r_j = jax.lax.broadcasted_iota(jnp.int32, (rows, LANES), 0)
    v_j = jax.lax.broadcasted_iota(jnp.int32, (rows, LANES), 1)
    a2 = ((r_j & 127) == v_j).astype(jnp.float32)
    t32 = jax.lax.dot_general(a1, t * a2, (((0,), (0,)), ((), ())),
                              precision=jax.lax.Precision.HIGHEST,
                              preferred_element_type=jnp.float32)

    strict_low = (jax.lax.broadcasted_iota(jnp.int32, (LANES, LANES), 0)
                  > jax.lax.broadcasted_iota(jnp.int32, (LANES, LANES), 1)
                  ).astype(jnp.float32)
    w_t = _dot(t32, strict_low)
    g_u = jnp.sum(t32, axis=1, keepdims=True)
    strict_up_u = (jax.lax.broadcasted_iota(jnp.int32, (u_dim, u_dim), 1)
                   > jax.lax.broadcasted_iota(jnp.int32, (u_dim, u_dim), 0)
                   ).astype(jnp.float32)
    a_u = _dot(strict_up_u, g_u)
    t_excl32 = w_t + a_u
    t_excl = jnp.sum(_dot(a1, t_excl32) * a2, axis=1, keepdims=True)

    s = s_row + t_excl
    loss = jnp.log(s + EPS_) - d
    out_ref[0, 0] = jnp.sum(loss) / (rows * LANES)


def _post_sort_loss(g2d):
    rows = g2d.shape[0]
    out = pl.pallas_call(
        _loss_body,
        out_shape=jax.ShapeDtypeStruct((1, 1), jnp.float32),
        in_specs=[pl.BlockSpec((rows, LANES), lambda: (0, 0))],
        out_specs=pl.BlockSpec((1, 1), lambda: (0, 0), memory_space=pltpu.SMEM),
    )(g2d)
    return out.reshape(())


_perm1 = _make_permute(PASS_SHIFTS[0], False, next_shift=PASS_SHIFTS[1])
_perm2 = _make_permute(PASS_SHIFTS[1], False, next_shift=PASS_SHIFTS[2])
_perm3 = _make_permute(PASS_SHIFTS[2], True)


@jax.jit
def kernel(y_pred, y_true):
    k0, h1 = _sc_keys(y_true)
    pre1, tot1 = _offsets(h1)
    k1, v1, h2 = _perm1(k0, y_pred, pre1, tot1)
    pre2, tot2 = _offsets(h2)
    k2, v2, h3 = _perm2(k1, v1, pre2, tot2)
    pre3, tot3 = _offsets(h3)
    (g_u,) = _perm3(k2, v2, pre3, tot3)
    return _post_sort_loss(g_u)


# speedup vs baseline: 3.0502x; 1.0301x over previous
import functools

import jax
import jax.numpy as jnp
from jax import lax
from jax.experimental import pallas as pl
from jax.experimental.pallas import tpu as pltpu
from jax.experimental.pallas import tpu_sc as plsc

EPS_ = 1e-05
LANES = 128
N_ = 1048576
CN_ = N_ // 2
VW = 8192
NVW = N_ // VW
TRASH = 8192
PASS_SHIFTS = (0, 11, 22)
DMASK = 2047

_mesh = plsc.VectorSubcoreMesh(core_axis_name="c", subcore_axis_name="s")
_CP = pltpu.CompilerParams(needs_layout_passes=False)


def _digit_u(k, shift):
    return ((k >> jnp.uint32(shift)) & jnp.uint32(DMASK)).astype(jnp.int32)


def _digit(kf, shift):
    return _digit_u(plsc.bitcast(kf, jnp.uint32), shift)


@functools.partial(
    pl.kernel,
    out_type=[jax.ShapeDtypeStruct((N_,), jnp.float32),
              jax.ShapeDtypeStruct((NVW, 2048), jnp.int32)],
    mesh=_mesh,
    scratch_types=[
        pltpu.VMEM((VW,), jnp.float32),
        pltpu.VMEM((VW,), jnp.float32),
        pltpu.VMEM((2048,), jnp.int32),
    ],
    compiler_params=_CP,
)
def _sc_keys(y_hbm, k_hbm, h_hbm, ybuf, kbuf, tb):
    c = lax.axis_index("c")
    s = lax.axis_index("s")
    wid = s * 2 + c

    for u in range(4):
        vw = wid * 4 + u
        pltpu.sync_copy(y_hbm.at[pl.ds(vw * VW, VW)], ybuf)

        def zero_body(i, _):
            tb[pl.ds(i * 16, 16)] = jnp.zeros((16,), jnp.int32)
            return 0

        lax.fori_loop(0, 2048 // 16, zero_body, 0, unroll=4)

        def body(i, _):
            sl = pl.ds(i * 16, 16)
            y = ybuf[sl]
            bu = plsc.bitcast(y, jnp.uint32)
            neg = plsc.bitcast(y, jnp.int32) < 0
            ku = jnp.where(neg, bu, jnp.uint32(0x7FFFFFFF) - bu)
            kbuf[sl] = plsc.bitcast(ku, jnp.float32)
            d = _digit_u(ku, 0)
            cnt, lastm = plsc.scan_count(d)
            plsc.addupdate_scatter(tb, [d], cnt, mask=lastm)
            return 0

        lax.fori_loop(0, VW // 16, body, 0, unroll=2)
        pltpu.sync_copy(kbuf, k_hbm.at[pl.ds(vw * VW, VW)])
        pltpu.sync_copy(tb, h_hbm.at[vw])


def _make_hist(shift):
    @functools.partial(
        pl.kernel,
        out_type=jax.ShapeDtypeStruct((NVW, 2048), jnp.int32),
        mesh=_mesh,
        scratch_types=[
            pltpu.VMEM((VW,), jnp.float32),
            pltpu.VMEM((2048,), jnp.int32),
        ],
        compiler_params=_CP,
    )
    def _hist(k_hbm, h_hbm, kbuf, tb):
        c = lax.axis_index("c")
        s = lax.axis_index("s")

        for u in range(4):
            vw = c * 64 + s * 4 + u
            pltpu.sync_copy(k_hbm.at[pl.ds(vw * VW, VW)], kbuf)

            def zero_body(i, _):
                tb[pl.ds(i * 16, 16)] = jnp.zeros((16,), jnp.int32)
                return 0

            lax.fori_loop(0, 2048 // 16, zero_body, 0, unroll=4)

            def body(i, _):
                d = _digit(kbuf[pl.ds(i * 16, 16)], shift)
                cnt, lastm = plsc.scan_count(d)
                plsc.addupdate_scatter(tb, [d], cnt, mask=lastm)
                return 0

            lax.fori_loop(0, VW // 16, body, 0, unroll=2)
            pltpu.sync_copy(tb, h_hbm.at[vw])

    return _hist


def _make_permute(shift, last_pass, next_shift=None):
    CH = VW // 4
    if last_pass:
        out_type = [jax.ShapeDtypeStruct((N_ // LANES, LANES), jnp.float32)]
    else:
        out_type = [jax.ShapeDtypeStruct((N_,), jnp.float32),
                    jax.ShapeDtypeStruct((N_,), jnp.float32),
                    jax.ShapeDtypeStruct((NVW, 2048), jnp.int32)]

    if last_pass:
        scratch = (
            [pltpu.VMEM((CH,), jnp.float32) for _ in range(16)]
            + [pltpu.VMEM((CH,), jnp.int32) for _ in range(8)]
            + [pltpu.VMEM((2048,), jnp.int32) for _ in range(4)]
            + [pltpu.VMEM((2048,), jnp.int32)]
            + [pltpu.VMEM_SHARED((CN_ + TRASH,), jnp.float32),
               pltpu.SemaphoreType.DMA, pltpu.SemaphoreType.DMA]
        )
    else:
        scratch = (
            [pltpu.VMEM((CH,), jnp.float32) for _ in range(8)]
            + [pltpu.VMEM((CH,), jnp.int32) for _ in range(32)]
            + [pltpu.VMEM((2048,), jnp.int32) for _ in range(4)]
            + [pltpu.VMEM((2048,), jnp.int32)]
            + [pltpu.VMEM_SHARED((CN_ + TRASH,), jnp.float32),
               pltpu.SemaphoreType.DMA, pltpu.SemaphoreType.DMA]
        )

    @functools.partial(
        pl.kernel,
        out_type=out_type,
        mesh=_mesh,
        scratch_types=scratch,
        compiler_params=_CP,
    )
    def _permute(k_hbm, v_hbm, pre_hbm, tot_hbm, *rest):
        if last_pass:
            vout_hbm = rest[0]
            kout_hbm = hnext_hbm = None
            rest = rest[1:]
            kbs = rest[0:8]
            vbs = rest[8:16]
            pls = rest[16:24]
            ofs = rest[24:28]
            basev = rest[28]
            ssh = rest[29]
            sem = rest[30]
            sem2 = rest[31]
        else:
            kout_hbm, vout_hbm, hnext_hbm = rest[:3]
            rest = rest[3:]
            kbs = rest[0:8]
            vbs = None
            pls = rest[8:40]
            ofs = rest[40:44]
            basev = rest[44]
            ssh = rest[45]
            sem = rest[46]
            sem2 = rest[47]

        c = lax.axis_index("c")
        s = lax.axis_index("s")
        cu = c.astype(jnp.uint32)
        cn_u = jnp.uint32(CN_)
        per_tile = CN_ // 16
        lo = s * per_tile
        glo = c * CN_ + s * per_tile

        pltpu.sync_copy(tot_hbm.at[0], ofs[0])

        def base_body(i, carry):
            sl = pl.ds(i * 16, 16)
            t = ofs[0][sl]
            incl = plsc.cumsum(t)
            basev[sl] = incl - t + carry
            return carry + jnp.sum(t)

        lax.fori_loop(0, 2048 // 16, base_body, jnp.int32(0))

        def src_off(p, u):
            vw = s * 8 + (p // 4) * 4 + u
            return vw * VW + (p % 4) * CH

        def start_loads(src_hbm, p):
            objs = []
            for u in range(4):
                cpo = pltpu.make_async_copy(
                    src_hbm.at[pl.ds(src_off(p, u), CH)],
                    kbs[(p % 2) * 4 + u], sem2)
                cpo.start()
                objs.append(cpo)
            return objs

        def start_scats(p, srcs, plbank):
            objs = []
            for u in range(4):
                cpo = pltpu.make_async_copy(
                    srcs[(p % 2) * 4 + u], ssh.at[pls[plbank(p) + u]], sem)
                cpo.start()
                objs.append(cpo)
            return objs

        def start_vloads(p):
            objs = []
            for u in range(4):
                cpo = pltpu.make_async_copy(
                    v_hbm.at[pl.ds(src_off(p, u), CH)],
                    vbs[(p % 2) * 4 + u], sem2)
                cpo.start()
                objs.append(cpo)
            return objs

        def reload_offsets(p):
            for u in range(4):
                vw = s * 8 + (p // 4) * 4 + u
                pltpu.sync_copy(pre_hbm.at[vw], ofs[u])

            def init_off(i, _):
                sl = pl.ds(i * 16, 16)
                b = basev[sl]
                for u in range(4):
                    ofs[u][sl] = ofs[u][sl] + b
                return 0

            lax.fori_loop(0, 2048 // 16, init_off, 0, unroll=2)

        def make_scan(p, plbank):
            def body(i, _, p=p):
                sl = pl.ds(i * 16, 16)
                j = ((p % 4) * CH + i * 16
                     + lax.broadcasted_iota(jnp.int32, (16,), 0))
                trash = CN_ + j
                for u in range(4):
                    d = _digit(kbs[(p % 2) * 4 + u][sl], shift)
                    cnt, lastm = plsc.scan_count(d)
                    bse = plsc.load_gather(ofs[u], [d])
                    pos = bse + cnt
                    plsc.store_scatter(ofs[u], [d], pos, mask=lastm)
                    if last_pass:
                        pos = jnp.where(pos < CN_, pos,
                                        (N_ - 1 + CN_) - pos)
                    my = pos.astype(jnp.uint32) - cu * cn_u
                    valid = my < cn_u
                    pls[plbank(p) + u][sl] = jnp.where(
                        valid, my.astype(jnp.int32), trash)
                return 0

            lax.fori_loop(0, CH // 16, body, 0, unroll=2)

        if last_pass:
            plbank = lambda p: (p % 2) * 4
            loads = {0: start_loads(k_hbm, 0)}
            vloads = {0: start_vloads(0)}
            scats = {}
            for p in range(8):
                if p % 4 == 0:
                    reload_offsets(p)
                for cpo in loads[p]:
                    cpo.wait()
                for cpo in vloads[p]:
                    cpo.wait()
                make_scan(p, plbank)
                scats[p] = start_scats(p, vbs, plbank)
                if p - 1 in scats:
                    for cpo in scats[p - 1]:
                        cpo.wait()
                if p + 1 < 8:
                    loads[p + 1] = start_loads(k_hbm, p + 1)
                    vloads[p + 1] = start_vloads(p + 1)
            for cpo in scats[7]:
                cpo.wait()

        if not last_pass:
            plbank = lambda p: p * 4
            loads = {0: start_loads(k_hbm, 0)}
            scats = {}
            for p in range(8):
                if p % 4 == 0:
                    reload_offsets(p)
                for cpo in loads[p]:
                    cpo.wait()
                make_scan(p, plbank)
                scats[p] = start_scats(p, kbs, plbank)
                if p - 1 in scats:
                    for cpo in scats[p - 1]:
                        cpo.wait()
                if p + 1 < 8:
                    loads[p + 1] = start_loads(k_hbm, p + 1)
            for cpo in scats[7]:
                cpo.wait()
            plsc.subcore_barrier()
            pltpu.sync_copy(ssh.at[pl.ds(lo, per_tile)],
                            kout_hbm.at[pl.ds(glo, per_tile)])

            def zero_body(i, _):
                sl = pl.ds(i * 16, 16)
                z = jnp.zeros((16,), jnp.int32)
                for u in range(4):
                    ofs[u][sl] = z
                return 0

            lax.fori_loop(0, 2048 // 16, zero_body, 0, unroll=2)
            for u in range(4):
                for h in range(4):
                    pltpu.sync_copy(
                        ssh.at[pl.ds(lo + u * VW + h * CH, CH)], kbs[u])

                    def hist_body(i, _, u=u):
                        d = _digit(kbs[u][pl.ds(i * 16, 16)], next_shift)
                        cnt, lastm = plsc.scan_count(d)
                        plsc.addupdate_scatter(ofs[u], [d], cnt, mask=lastm)
                        return 0

                    lax.fori_loop(0, CH // 16, hist_body, 0, unroll=2)
            vw0 = glo // VW
            for u in range(4):
                pltpu.sync_copy(ofs[u], hnext_hbm.at[vw0 + u])

            plsc.subcore_barrier()

        if not last_pass:
            loads = {0: start_loads(v_hbm, 0)}
            scats = {}
            for p in range(8):
                for cpo in loads[p]:
                    cpo.wait()
                scats[p] = start_scats(p, kbs, lambda p: p * 4)
                if p - 1 in scats:
                    for cpo in scats[p - 1]:
                        cpo.wait()
                if p + 1 < 8:
                    loads[p + 1] = start_loads(v_hbm, p + 1)
            for cpo in scats[7]:
                cpo.wait()

        plsc.subcore_barrier()
        if last_pass:
            row0 = glo // LANES
            copies = [
                pltpu.make_async_copy(
                    ssh.at[pl.ds(lo + r * LANES, LANES)],
                    vout_hbm.at[row0 + r], sem)
                for r in range(per_tile // LANES)]
            for cp in copies:
                cp.start()
            for cp in copies:
                cp.wait()
        else:
            pltpu.sync_copy(ssh.at[pl.ds(lo, per_tile)],
                            vout_hbm.at[pl.ds(glo, per_tile)])

    return _permute


def _dot(x, y):
    return jax.lax.dot(x, y, precision=jax.lax.Precision.HIGHEST,
                       preferred_element_type=jnp.float32)


def _offsets_body(h_ref, pre_ref, tot_ref):
    hf = h_ref[...].astype(jnp.float32)
    nw = hf.shape[0]
    strict_low_w = (jax.lax.broadcasted_iota(jnp.int32, (nw, nw), 1)
                    < jax.lax.broadcasted_iota(jnp.int32, (nw, nw), 0)
                    ).astype(jnp.float32)
    pre = _dot(strict_low_w, hf)
    ones8 = jnp.ones((8, nw), jnp.float32)
    tot = _dot(ones8, hf)
    pre_ref[...] = (pre - 1.0).astype(jnp.int32)
    tot_ref[...] = tot.astype(jnp.int32)


def _offsets(h):
    return pl.pallas_call(
        _offsets_body,
        out_shape=[jax.ShapeDtypeStruct((NVW, 2048), jnp.int32),
                   jax.ShapeDtypeStruct((8, 2048), jnp.int32)],
        in_specs=[pl.BlockSpec((NVW, 2048), lambda: (0, 0))],
        out_specs=[pl.BlockSpec((NVW, 2048), lambda: (0, 0)),
                   pl.BlockSpec((8, 2048), lambda: (0, 0))],
    )(h)


def _loss_body(g_ref, out_ref):
    full = g_ref[...]
    half = full.shape[0] // 2
    a = full[:half, :]
    b = full[half:, :]
    rows = a.shape[0]
    u_dim = rows // LANES
    d = a - b
    e = jnp.exp(d)

    lower_incl = (jax.lax.broadcasted_iota(jnp.int32, (LANES, LANES), 0)
                  >= jax.lax.broadcasted_iota(jnp.int32, (LANES, LANES), 1)
                  ).astype(jnp.float32)
    s_row = _dot(e, lower_incl)

    t = jnp.sum(e, axis=1, keepdims=True)
    r_i = jax.lax.broadcasted_iota(jnp.int32, (rows, u_dim), 0)
    u_i = jax.lax.broadcasted_iota(jnp.int32, (rows, u_dim), 1)
    a1 = ((r_i >> 7) == u_i).astype(jnp.float32)
    r_j = jax.lax.broadcasted_iota(jnp.int32, (rows, LANES), 0)
    v_j = jax.lax.broadcasted_iota(jnp.int32, (rows, LANES), 1)
    a2 = ((r_j & 127) == v_j).astype(jnp.float32)
    t32 = jax.lax.dot_general(a1, t * a2, (((0,), (0,)), ((), ())),
                              precision=jax.lax.Precision.HIGHEST,
                              preferred_element_type=jnp.float32)

    strict_low = (jax.lax.broadcasted_iota(jnp.int32, (LANES, LANES), 0)
                  > jax.lax.broadcasted_iota(jnp.int32, (LANES, LANES), 1)
                  ).astype(jnp.float32)
    w_t = _dot(t32, strict_low)
    g_u = jnp.sum(t32, axis=1, keepdims=True)
    strict_up_u = (jax.lax.broadcasted_iota(jnp.int32, (u_dim, u_dim), 1)
                   > jax.lax.broadcasted_iota(jnp.int32, (u_dim, u_dim), 0)
                   ).astype(jnp.float32)
    a_u = _dot(strict_up_u, g_u)
    t_excl32 = w_t + a_u
    t_excl = jnp.sum(_dot(a1, t_excl32) * a2, axis=1, keepdims=True)

    s = s_row + t_excl
    loss = jnp.log(s + EPS_) - d
    out_ref[0, 0] = jnp.sum(loss) / (rows * LANES)


def _post_sort_loss(g2d):
    rows = g2d.shape[0]
    out = pl.pallas_call(
        _loss_body,
        out_shape=jax.ShapeDtypeStruct((1, 1), jnp.float32),
        in_specs=[pl.BlockSpec((rows, LANES), lambda: (0, 0))],
        out_specs=pl.BlockSpec((1, 1), lambda: (0, 0), memory_space=pltpu.SMEM),
    )(g2d)
    return out.reshape(())


_perm1 = _make_permute(PASS_SHIFTS[0], False, next_shift=PASS_SHIFTS[1])
_perm2 = _make_permute(PASS_SHIFTS[1], False, next_shift=PASS_SHIFTS[2])
_perm3 = _make_permute(PASS_SHIFTS[2], True)


@jax.jit
def kernel(y_pred, y_true):
    k0, h1 = _sc_keys(y_true)
    pre1, tot1 = _offsets(h1)
    k1, v1, h2 = _perm1(k0, y_pred, pre1, tot1)
    pre2, tot2 = _offsets(h2)
    k2, v2, h3 = _perm2(k1, v1, pre2, tot2)
    pre3, tot3 = _offsets(h3)
    (g_u,) = _perm3(k2, v2, pre3, tot3)
    return _post_sort_loss(g_u)


# speedup vs baseline: 3.0527x; 1.0008x over previous
import functools

import jax
import jax.numpy as jnp
from jax import lax
from jax.experimental import pallas as pl
from jax.experimental.pallas import tpu as pltpu
from jax.experimental.pallas import tpu_sc as plsc

EPS_ = 1e-05
LANES = 128
N_ = 1048576
CN_ = N_ // 2
VW = 8192
NVW = N_ // VW
TRASH = 8192
PASS_SHIFTS = (0, 11, 22)
DMASK = 2047

_mesh = plsc.VectorSubcoreMesh(core_axis_name="c", subcore_axis_name="s")
_CP = pltpu.CompilerParams(needs_layout_passes=False)


def _digit_u(k, shift):
    return ((k >> jnp.uint32(shift)) & jnp.uint32(DMASK)).astype(jnp.int32)


def _digit(kf, shift):
    return _digit_u(plsc.bitcast(kf, jnp.uint32), shift)


@functools.partial(
    pl.kernel,
    out_type=[jax.ShapeDtypeStruct((N_,), jnp.float32),
              jax.ShapeDtypeStruct((NVW, 2048), jnp.int32)],
    mesh=_mesh,
    scratch_types=[
        pltpu.VMEM((VW,), jnp.float32),
        pltpu.VMEM((VW,), jnp.float32),
        pltpu.VMEM((2048,), jnp.int32),
    ],
    compiler_params=_CP,
)
def _sc_keys(y_hbm, k_hbm, h_hbm, ybuf, kbuf, tb):
    c = lax.axis_index("c")
    s = lax.axis_index("s")
    wid = s * 2 + c

    for u in range(4):
        vw = wid * 4 + u
        pltpu.sync_copy(y_hbm.at[pl.ds(vw * VW, VW)], ybuf)

        def zero_body(i, _):
            tb[pl.ds(i * 16, 16)] = jnp.zeros((16,), jnp.int32)
            return 0

        lax.fori_loop(0, 2048 // 16, zero_body, 0, unroll=4)

        def body(i, _):
            sl = pl.ds(i * 16, 16)
            y = ybuf[sl]
            bu = plsc.bitcast(y, jnp.uint32)
            neg = plsc.bitcast(y, jnp.int32) < 0
            ku = jnp.where(neg, bu, jnp.uint32(0x7FFFFFFF) - bu)
            kbuf[sl] = plsc.bitcast(ku, jnp.float32)
            d = _digit_u(ku, 0)
            cnt, lastm = plsc.scan_count(d)
            plsc.addupdate_scatter(tb, [d], cnt, mask=lastm)
            return 0

        lax.fori_loop(0, VW // 16, body, 0, unroll=2)
        pltpu.sync_copy(kbuf, k_hbm.at[pl.ds(vw * VW, VW)])
        pltpu.sync_copy(tb, h_hbm.at[vw])


def _make_permute(shift, last_pass, next_shift=None):
    CH = VW // 4
    if last_pass:
        out_type = [jax.ShapeDtypeStruct((N_ // LANES, LANES), jnp.float32)]
    else:
        out_type = [jax.ShapeDtypeStruct((N_,), jnp.float32),
                    jax.ShapeDtypeStruct((N_,), jnp.float32),
                    jax.ShapeDtypeStruct((NVW, 2048), jnp.int32)]

    if last_pass:
        scratch = (
            [pltpu.VMEM((CH,), jnp.float32) for _ in range(16)]
            + [pltpu.VMEM((CH,), jnp.int32) for _ in range(8)]
            + [pltpu.VMEM((2048,), jnp.int32) for _ in range(4)]
            + [pltpu.VMEM((2048,), jnp.int32)]
            + [pltpu.VMEM_SHARED((CN_ + TRASH,), jnp.float32),
               pltpu.SemaphoreType.DMA, pltpu.SemaphoreType.DMA]
        )
    else:
        scratch = (
            [pltpu.VMEM((CH,), jnp.float32) for _ in range(8)]
            + [pltpu.VMEM((CH,), jnp.int32) for _ in range(32)]
            + [pltpu.VMEM((2048,), jnp.int32) for _ in range(4)]
            + [pltpu.VMEM((2048,), jnp.int32)]
            + [pltpu.VMEM_SHARED((CN_ + TRASH,), jnp.float32),
               pltpu.SemaphoreType.DMA, pltpu.SemaphoreType.DMA]
        )

    @functools.partial(
        pl.kernel,
        out_type=out_type,
        mesh=_mesh,
        scratch_types=scratch,
        compiler_params=_CP,
    )
    def _permute(k_hbm, v_hbm, pre_hbm, tot_hbm, *rest):
        if last_pass:
            vout_hbm = rest[0]
            kout_hbm = hnext_hbm = None
            rest = rest[1:]
            kbs = rest[0:8]
            vbs = rest[8:16]
            pls = rest[16:24]
            ofs = rest[24:28]
            basev = rest[28]
            ssh = rest[29]
            sem = rest[30]
            sem2 = rest[31]
        else:
            kout_hbm, vout_hbm, hnext_hbm = rest[:3]
            rest = rest[3:]
            kbs = rest[0:8]
            vbs = None
            pls = rest[8:40]
            ofs = rest[40:44]
            basev = rest[44]
            ssh = rest[45]
            sem = rest[46]
            sem2 = rest[47]

        c = lax.axis_index("c")
        s = lax.axis_index("s")
        cu = c.astype(jnp.uint32)
        cn_u = jnp.uint32(CN_)
        per_tile = CN_ // 16
        lo = s * per_tile
        glo = c * CN_ + s * per_tile

        pltpu.sync_copy(tot_hbm.at[0], ofs[0])

        def base_body(i, carry):
            sl = pl.ds(i * 16, 16)
            t = ofs[0][sl]
            incl = plsc.cumsum(t)
            basev[sl] = incl - t + carry
            return carry + jnp.sum(t)

        lax.fori_loop(0, 2048 // 16, base_body, jnp.int32(0))

        def src_off(p, u):
            vw = s * 8 + (p // 4) * 4 + u
            return vw * VW + (p % 4) * CH

        def start_loads(src_hbm, p):
            objs = []
            for u in range(4):
                cpo = pltpu.make_async_copy(
                    src_hbm.at[pl.ds(src_off(p, u), CH)],
                    kbs[(p % 2) * 4 + u], sem2)
                cpo.start()
                objs.append(cpo)
            return objs

        def start_scats(p, srcs, plbank):
            objs = []
            for u in range(4):
                cpo = pltpu.make_async_copy(
                    srcs[(p % 2) * 4 + u], ssh.at[pls[plbank(p) + u]], sem)
                cpo.start()
                objs.append(cpo)
            return objs

        def start_vloads(p):
            objs = []
            for u in range(4):
                cpo = pltpu.make_async_copy(
                    v_hbm.at[pl.ds(src_off(p, u), CH)],
                    vbs[(p % 2) * 4 + u], sem2)
                cpo.start()
                objs.append(cpo)
            return objs

        def reload_offsets(p):
            for u in range(4):
                vw = s * 8 + (p // 4) * 4 + u
                pltpu.sync_copy(pre_hbm.at[vw], ofs[u])

            def init_off(i, _):
                sl = pl.ds(i * 16, 16)
                b = basev[sl]
                for u in range(4):
                    ofs[u][sl] = ofs[u][sl] + b
                return 0

            lax.fori_loop(0, 2048 // 16, init_off, 0, unroll=2)

        def make_scan(p, plbank):
            def body(i, _, p=p):
                sl = pl.ds(i * 16, 16)
                j = ((p % 4) * CH + i * 16
                     + lax.broadcasted_iota(jnp.int32, (16,), 0))
                trash = CN_ + j
                for u in range(4):
                    d = _digit(kbs[(p % 2) * 4 + u][sl], shift)
                    cnt, lastm = plsc.scan_count(d)
                    bse = plsc.load_gather(ofs[u], [d])
                    pos = bse + cnt
                    plsc.store_scatter(ofs[u], [d], pos, mask=lastm)
                    if last_pass:
                        pos = jnp.where(pos < CN_, pos,
                                        (N_ - 1 + CN_) - pos)
                    my = pos.astype(jnp.uint32) - cu * cn_u
                    valid = my < cn_u
                    pls[plbank(p) + u][sl] = jnp.where(
                        valid, my.astype(jnp.int32), trash)
                return 0

            lax.fori_loop(0, CH // 16, body, 0, unroll=2)

        if last_pass:
            plbank = lambda p: (p % 2) * 4
            loads = {0: start_loads(k_hbm, 0)}
            vloads = {0: start_vloads(0)}
            scats = {}
            for p in range(8):
                if p % 4 == 0:
                    reload_offsets(p)
                for cpo in loads[p]:
                    cpo.wait()
                for cpo in vloads[p]:
                    cpo.wait()
                make_scan(p, plbank)
                scats[p] = start_scats(p, vbs, plbank)
                if p - 1 in scats:
                    for cpo in scats[p - 1]:
                        cpo.wait()
                if p + 1 < 8:
                    loads[p + 1] = start_loads(k_hbm, p + 1)
                    vloads[p + 1] = start_vloads(p + 1)
            for cpo in scats[7]:
                cpo.wait()

        if not last_pass:
            plbank = lambda p: p * 4
            loads = {0: start_loads(k_hbm, 0)}
            scats = {}
            for p in range(8):
                if p % 4 == 0:
                    reload_offsets(p)
                for cpo in loads[p]:
                    cpo.wait()
                make_scan(p, plbank)
                scats[p] = start_scats(p, kbs, plbank)
                if p - 1 in scats:
                    for cpo in scats[p - 1]:
                        cpo.wait()
                if p + 1 < 8:
                    loads[p + 1] = start_loads(k_hbm, p + 1)
            for cpo in scats[7]:
                cpo.wait()
            plsc.subcore_barrier()
            pltpu.sync_copy(ssh.at[pl.ds(lo, per_tile)],
                            kout_hbm.at[pl.ds(glo, per_tile)])

            def zero_body(i, _):
                sl = pl.ds(i * 16, 16)
                z = jnp.zeros((16,), jnp.int32)
                for u in range(4):
                    ofs[u][sl] = z
                return 0

            lax.fori_loop(0, 2048 // 16, zero_body, 0, unroll=2)
            for u in range(4):
                for h in range(4):
                    pltpu.sync_copy(
                        ssh.at[pl.ds(lo + u * VW + h * CH, CH)], kbs[u])

                    def hist_body(i, _, u=u):
                        d = _digit(kbs[u][pl.ds(i * 16, 16)], next_shift)
                        cnt, lastm = plsc.scan_count(d)
                        plsc.addupdate_scatter(ofs[u], [d], cnt, mask=lastm)
                        return 0

                    lax.fori_loop(0, CH // 16, hist_body, 0, unroll=2)
            vw0 = glo // VW
            for u in range(4):
                pltpu.sync_copy(ofs[u], hnext_hbm.at[vw0 + u])

            plsc.subcore_barrier()

        if not last_pass:
            loads = {0: start_loads(v_hbm, 0)}
            scats = {}
            for p in range(8):
                for cpo in loads[p]:
                    cpo.wait()
                scats[p] = start_scats(p, kbs, lambda p: p * 4)
                if p - 1 in scats:
                    for cpo in scats[p - 1]:
                        cpo.wait()
                if p + 1 < 8:
                    loads[p + 1] = start_loads(v_hbm, p + 1)
            for cpo in scats[7]:
                cpo.wait()

        plsc.subcore_barrier()
        if last_pass:
            row0 = glo // LANES
            copies = [
                pltpu.make_async_copy(
                    ssh.at[pl.ds(lo + r * LANES, LANES)],
                    vout_hbm.at[row0 + r], sem)
                for r in range(per_tile // LANES)]
            for cp in copies:
                cp.start()
            for cp in copies:
                cp.wait()
        else:
            pltpu.sync_copy(ssh.at[pl.ds(lo, per_tile)],
                            vout_hbm.at[pl.ds(glo, per_tile)])

    return _permute


def _dot(x, y):
    return jax.lax.dot(x, y, precision=jax.lax.Precision.HIGHEST,
                       preferred_element_type=jnp.float32)


def _offsets_body(h_ref, pre_ref, tot_ref):
    hf = h_ref[...].astype(jnp.float32)
    nw = hf.shape[0]
    strict_low_w = (jax.lax.broadcasted_iota(jnp.int32, (nw, nw), 1)
                    < jax.lax.broadcasted_iota(jnp.int32, (nw, nw), 0)
                    ).astype(jnp.float32)
    pre = _dot(strict_low_w, hf)
    ones8 = jnp.ones((8, nw), jnp.float32)
    tot = _dot(ones8, hf)
    pre_ref[...] = (pre - 1.0).astype(jnp.int32)
    tot_ref[...] = tot.astype(jnp.int32)


def _offsets(h):
    return pl.pallas_call(
        _offsets_body,
        out_shape=[jax.ShapeDtypeStruct((NVW, 2048), jnp.int32),
                   jax.ShapeDtypeStruct((8, 2048), jnp.int32)],
        in_specs=[pl.BlockSpec((NVW, 2048), lambda: (0, 0))],
        out_specs=[pl.BlockSpec((NVW, 2048), lambda: (0, 0)),
                   pl.BlockSpec((8, 2048), lambda: (0, 0))],
    )(h)


def _loss_body(g_ref, out_ref):
    full = g_ref[...]
    half = full.shape[0] // 2
    a = full[:half, :]
    b = full[half:, :]
    rows = a.shape[0]
    u_dim = rows // LANES
    d = a - b
    e = jnp.exp(d)

    lower_incl = (jax.lax.broadcasted_iota(jnp.int32, (LANES, LANES), 0)
                  >= jax.lax.broadcasted_iota(jnp.int32, (LANES, LANES), 1)
                  ).astype(jnp.float32)
    s_row = _dot(e, lower_incl)

    t = jnp.sum(e, axis=1, keepdims=True)
    r_i = jax.lax.broadcasted_iota(jnp.int32, (rows, u_dim), 0)
    u_i = jax.lax.broadcasted_iota(jnp.int32, (rows, u_dim), 1)
    a1 = ((r_i >> 7) == u_i).astype(jnp.float32)
    r_j = jax.lax.broadcasted_iota(jnp.int32, (rows, LANES), 0)
    v_j = jax.lax.broadcasted_iota(jnp.int32, (rows, LANES), 1)
    a2 = ((r_j & 127) == v_j).astype(jnp.float32)
    t32 = jax.lax.dot_general(a1, t * a2, (((0,), (0,)), ((), ())),
                              precision=jax.lax.Precision.HIGHEST,
                              preferred_element_type=jnp.float32)

    strict_low = (jax.lax.broadcasted_iota(jnp.int32, (LANES, LANES), 0)
                  > jax.lax.broadcasted_iota(jnp.int32, (LANES, LANES), 1)
                  ).astype(jnp.float32)
    w_t = _dot(t32, strict_low)
    g_u = jnp.sum(t32, axis=1, keepdims=True)
    strict_up_u = (jax.lax.broadcasted_iota(jnp.int32, (u_dim, u_dim), 1)
                   > jax.lax.broadcasted_iota(jnp.int32, (u_dim, u_dim), 0)
                   ).astype(jnp.float32)
    a_u = _dot(strict_up_u, g_u)
    t_excl32 = w_t + a_u
    t_excl = jnp.sum(_dot(a1, t_excl32) * a2, axis=1, keepdims=True)

    s = s_row + t_excl
    loss = jnp.log(s + EPS_) - d
    out_ref[0, 0] = jnp.sum(loss) / (rows * LANES)


def _post_sort_loss(g2d):
    rows = g2d.shape[0]
    out = pl.pallas_call(
        _loss_body,
        out_shape=jax.ShapeDtypeStruct((1, 1), jnp.float32),
        in_specs=[pl.BlockSpec((rows, LANES), lambda: (0, 0))],
        out_specs=pl.BlockSpec((1, 1), lambda: (0, 0), memory_space=pltpu.SMEM),
    )(g2d)
    return out.reshape(())


_perm1 = _make_permute(PASS_SHIFTS[0], False, next_shift=PASS_SHIFTS[1])
_perm2 = _make_permute(PASS_SHIFTS[1], False, next_shift=PASS_SHIFTS[2])
_perm3 = _make_permute(PASS_SHIFTS[2], True)


@jax.jit
def kernel(y_pred, y_true):
    k0, h1 = _sc_keys(y_true)
    pre1, tot1 = _offsets(h1)
    k1, v1, h2 = _perm1(k0, y_pred, pre1, tot1)
    pre2, tot2 = _offsets(h2)
    k2, v2, h3 = _perm2(k1, v1, pre2, tot2)
    pre3, tot3 = _offsets(h3)
    (g_u,) = _perm3(k2, v2, pre3, tot3)
    return _post_sort_loss(g_u)
